# baseline (speedup 1.0000x reference)
import functools

import jax
import jax.numpy as jnp
from jax import lax
from jax.experimental import pallas as pl
from jax.experimental.pallas import tpu as pltpu
from jax.experimental.pallas import tpu_sc as plsc

HID = 128
EDIM = 16
PPAD = 16
MW = HID + EDIM
N_NODES = 10000
E_EDGES = 320000
EROWS = E_EDGES // 128
EPS = 1e-6

EDGE_BLK = 1280
NODE_BLK = 2000

_NC = 2
_NS = 16
_NW = _NC * _NS
_RB = EROWS // _NW
_XTRA = EROWS - _RB * _NW
_NPA = 10112
_NPT = _NPA // _NS


def _silu(x):
    return x * jax.nn.sigmoid(x)


def _ln_x(x, g, b):
    m = jnp.sum(x, axis=-1, keepdims=True) / HID
    xc = x - m
    v = jnp.sum(xc * xc, axis=-1, keepdims=True) / HID
    return xc / jnp.sqrt(v + EPS) * g + b


def _ln_pos_t(p, g, b):
    rowmask = (lax.broadcasted_iota(jnp.int32, (4, 1), 0) < 3).astype(jnp.float32)
    m = jnp.sum(p, axis=0, keepdims=True) / 3.0
    pc = p - m
    pcm = pc * rowmask
    v = jnp.sum(pcm * pcm, axis=0, keepdims=True) / 3.0
    return pc / jnp.sqrt(v + EPS) * g + b


def _prep_body(x_ref, pt_ref, aux_ref, t_ref, p_ref):
    t_ref[...] = _ln_x(x_ref[...], aux_ref[1, :], aux_ref[2, :])
    p_ref[...] = _ln_pos_t(pt_ref[...], aux_ref[4:8, 0:1], aux_ref[4:8, 1:2])


def _edge_body(g_ref, rel_ref, attr_ref, w1_ref, sw2_ref, aux_ref,
               s_out_ref, v_out_ref):
    xj = g_ref[...]
    attr = attr_ref[...]

    ca = aux_ref[0, :]
    a_b1 = aux_ref[1, :]
    bs = aux_ref[2, :]
    bv = aux_ref[3, :]
    a_w2 = aux_ref[4, :]
    v_w2 = aux_ref[5, :]
    s_b2 = aux_ref[6, :]
    s2g = aux_ref[7, :]
    s2b = aux_ref[8, :]
    a_b2 = aux_ref[9, 0]
    v_b2 = aux_ref[9, 1]

    s1 = jnp.sum(xj, axis=-1, keepdims=True) + jnp.sum(attr, axis=-1, keepdims=True)
    m = s1 / MW
    xc = xj - m
    ac = attr - m
    var = (jnp.sum(xc * xc, axis=-1, keepdims=True)
           + jnp.sum(ac * ac, axis=-1, keepdims=True)) / MW
    sd = jnp.sqrt(var + EPS)
    inv = 1.0 / sd
    n = jnp.concatenate([xc * inv, ac * inv], axis=1)

    pre = jnp.dot(n, w1_ref[...], preferred_element_type=jnp.float32)
    pre_a = sd * pre[:, :HID] + m * ca + a_b1
    pre_s = pre[:, HID:2 * HID] + bs
    pre_v = pre[:, 2 * HID:] + bv

    a = jnp.sum(_silu(pre_a) * a_w2, axis=-1, keepdims=True) + a_b2
    attn = jax.nn.sigmoid(a)

    h = jnp.dot(_silu(pre_s), sw2_ref[...], preferred_element_type=jnp.float32) + s_b2
    h = _ln_x(h, s2g, s2b)
    s_out_ref[...] = h * attn

    rel_t = rel_ref[...]
    dist = jnp.maximum(
        jnp.sqrt(jnp.sum(rel_t * rel_t, axis=0, keepdims=True)), 1e-6)
    dims = (((0,), (1,)), ((), ()))
    a_row = lax.dot_general(a_w2[:, None], _silu(pre_a), dims,
                            preferred_element_type=jnp.float32) + a_b2
    vw_row = lax.dot_general(v_w2[:, None], _silu(pre_v), dims,
                             preferred_element_type=jnp.float32) + v_b2
    v_out_ref[...] = rel_t * (vw_row * jax.nn.sigmoid(a_row) / dist)


def _node_core(t_ref, pt_ref, p0_ref, p1_ref, vp_ref, gw_ref, gb):
    xln = t_ref[...]
    s_agg = p0_ref[...] + p1_ref[...]
    v_agg = jnp.sum(vp_ref[...], axis=0)
    v_agg = jnp.concatenate(
        [v_agg, jnp.zeros((1, v_agg.shape[1]), jnp.float32)], axis=0)
    gate = jax.nn.sigmoid(
        jnp.dot(xln, gw_ref[:HID, :], preferred_element_type=jnp.float32)
        + jnp.dot(s_agg, gw_ref[HID:, :], preferred_element_type=jnp.float32)
        + gb)
    x_new = xln * (1.0 - gate) + s_agg * gate
    pos_new = jnp.clip(pt_ref[...] + v_agg, -10.0, 10.0)
    return x_new, pos_new


def _update_body(t_ref, pt_ref, p0_ref, p1_ref, vp_ref, gw_ref, aux_ref,
                 t_out_ref, p_out_ref):
    x_new, pos_new = _node_core(t_ref, pt_ref, p0_ref, p1_ref, vp_ref, gw_ref,
                                aux_ref[0, :])
    t_out_ref[...] = _ln_x(x_new, aux_ref[1, :], aux_ref[2, :])
    p_out_ref[...] = _ln_pos_t(pos_new, aux_ref[4:8, 0:1], aux_ref[4:8, 1:2])


def _final_body(t_ref, pt_ref, p0_ref, p1_ref, vp_ref, gw_ref, ew1_ref,
                ew2_ref, aux_ref, x_out_ref, p_out_ref):
    x_new, pos_new = _node_core(t_ref, pt_ref, p0_ref, p1_ref, vp_ref, gw_ref,
                                aux_ref[0, :])
    y = jax.nn.relu(
        jnp.dot(x_new, ew1_ref[...], preferred_element_type=jnp.float32)
        + aux_ref[1, :])
    y = jnp.dot(y, ew2_ref[...], preferred_element_type=jnp.float32) + aux_ref[2, :]
    x_out_ref[...] = y
    p_out_ref[...] = pos_new


def _tc_prep(x, pos_t, aux):
    return pl.pallas_call(
        _prep_body,
        out_shape=[
            jax.ShapeDtypeStruct((N_NODES, HID), jnp.float32),
            jax.ShapeDtypeStruct((4, N_NODES), jnp.float32),
        ],
    )(x, pos_t, aux)


def _tc_edge(gat, rel, attr, w1, sw2, aux):
    grid = E_EDGES // EDGE_BLK
    return pl.pallas_call(
        _edge_body,
        grid=(grid,),
        in_specs=[
            pl.BlockSpec((EDGE_BLK, HID), lambda i: (i, 0)),
            pl.BlockSpec((3, EDGE_BLK), lambda i: (0, i)),
            pl.BlockSpec((EDGE_BLK, EDIM), lambda i: (i, 0)),
            pl.BlockSpec((MW, 3 * HID), lambda i: (0, 0)),
            pl.BlockSpec((HID, HID), lambda i: (0, 0)),
            pl.BlockSpec((16, HID), lambda i: (0, 0)),
        ],
        out_specs=[
            pl.BlockSpec((EDGE_BLK, HID), lambda i: (i, 0)),
            pl.BlockSpec((3, EDGE_BLK), lambda i: (0, i)),
        ],
        out_shape=[
            jax.ShapeDtypeStruct((E_EDGES, HID), jnp.float32),
            jax.ShapeDtypeStruct((3, E_EDGES), jnp.float32),
        ],
    )(gat, rel, attr, w1, sw2, aux)


def _tc_update(t, pt, sp, vp, gw, aux):
    return pl.pallas_call(
        _update_body,
        grid=(1,),
        in_specs=[
            pl.BlockSpec((N_NODES, HID), lambda i: (0, 0)),
            pl.BlockSpec((4, N_NODES), lambda i: (0, 0)),
            pl.BlockSpec((None, N_NODES, HID), lambda i: (0, 0, 0)),
            pl.BlockSpec((None, N_NODES, HID), lambda i: (1, 0, 0)),
            pl.BlockSpec((_NW, 3, N_NODES), lambda i: (0, 0, 0)),
            pl.BlockSpec((2 * HID, HID), lambda i: (0, 0)),
            pl.BlockSpec((8, HID), lambda i: (0, 0)),
        ],
        out_specs=[
            pl.BlockSpec((N_NODES, HID), lambda i: (0, 0)),
            pl.BlockSpec((4, N_NODES), lambda i: (0, 0)),
        ],
        out_shape=[
            jax.ShapeDtypeStruct((N_NODES, HID), jnp.float32),
            jax.ShapeDtypeStruct((4, N_NODES), jnp.float32),
        ],
    )(t, pt, sp, sp, vp, gw, aux)


def _tc_final(t, pt, sp, vp, gw, ew1, ew2, aux):
    return pl.pallas_call(
        _final_body,
        grid=(1,),
        in_specs=[
            pl.BlockSpec((N_NODES, HID), lambda i: (0, 0)),
            pl.BlockSpec((4, N_NODES), lambda i: (0, 0)),
            pl.BlockSpec((None, N_NODES, HID), lambda i: (0, 0, 0)),
            pl.BlockSpec((None, N_NODES, HID), lambda i: (1, 0, 0)),
            pl.BlockSpec((_NW, 3, N_NODES), lambda i: (0, 0, 0)),
            pl.BlockSpec((2 * HID, HID), lambda i: (0, 0)),
            pl.BlockSpec((HID, HID), lambda i: (0, 0)),
            pl.BlockSpec((HID, HID), lambda i: (0, 0)),
            pl.BlockSpec((4, HID), lambda i: (0, 0)),
        ],
        out_specs=[
            pl.BlockSpec((N_NODES, HID), lambda i: (0, 0)),
            pl.BlockSpec((4, N_NODES), lambda i: (0, 0)),
        ],
        out_shape=[
            jax.ShapeDtypeStruct((N_NODES, HID), jnp.float32),
            jax.ShapeDtypeStruct((4, N_NODES), jnp.float32),
        ],
    )(t, pt, sp, sp, vp, gw, ew1, ew2, aux)


def _worker_range(w):
    start = jnp.where(w < _XTRA, w * (_RB + 1), _XTRA * (_RB + 1) + (w - _XTRA) * _RB)
    cnt = jnp.where(w < _XTRA, _RB + 1, _RB)
    return start, cnt


def _sc_gather_body(t_hbm, pf_hbm, src_hbm, dst_hbm, g_hbm, relt_hbm,
                    sidx_v, didx_v, rows_a, rows_b, rbuf_a, rbuf_b, posf_v,
                    gs_a, gs_b, ss_a, ss_b, rs_a, rs_b):
    w = lax.axis_index("s") * _NC + lax.axis_index("c")
    start = w * 80
    cnt = jnp.minimum(80, EROWS - start)

    pltpu.sync_copy(pf_hbm, posf_v)
    pltpu.sync_copy(src_hbm.at[pl.ds(start, 80)], sidx_v)
    pltpu.sync_copy(dst_hbm.at[pl.ds(start, 80)], didx_v)

    def rel_compute(i, rbuf):
        for g in range(8):
            si = sidx_v[i, pl.ds(g * 16, 16)]
            di = didx_v[i, pl.ds(g * 16, 16)]
            for d in range(3):
                off = jnp.full((16,), d * N_NODES, jnp.int32)
                ps = plsc.load_gather(posf_v, [si + off])
                pd = plsc.load_gather(posf_v, [di + off])
                rbuf[d, pl.ds(g * 16, 16)] = ps - pd

    pltpu.async_copy(t_hbm.at[sidx_v.at[0]], rows_a, gs_a)

    def pair(jj, carry):
        i0 = 2 * jj
        i1 = i0 + 1
        r0 = start + i0
        r1 = r0 + 1

        @pl.when(jj > 0)
        def _():
            pltpu.make_async_copy(
                rows_b, g_hbm.at[pl.ds((r0 - 1) * 128, 128)], ss_b).wait()
            pltpu.make_async_copy(
                rbuf_b, relt_hbm.at[:, pl.ds((r0 - 1) * 128, 128)], rs_b).wait()

        pltpu.async_copy(t_hbm.at[sidx_v.at[i1]], rows_b, gs_b)
        pltpu.make_async_copy(t_hbm.at[sidx_v.at[i0]], rows_a, gs_a).wait()
        rel_compute(i0, rbuf_a)
        pltpu.async_copy(rows_a, g_hbm.at[pl.ds(r0 * 128, 128)], ss_a)
        pltpu.async_copy(rbuf_a, relt_hbm.at[:, pl.ds(r0 * 128, 128)], rs_a)
        pltpu.make_async_copy(t_hbm.at[sidx_v.at[i1]], rows_b, gs_b).wait()
        rel_compute(i1, rbuf_b)
        pltpu.make_async_copy(
            rows_a, g_hbm.at[pl.ds(r0 * 128, 128)], ss_a).wait()
        pltpu.make_async_copy(
            rbuf_a, relt_hbm.at[:, pl.ds(r0 * 128, 128)], rs_a).wait()
        pltpu.async_copy(rows_b, g_hbm.at[pl.ds(r1 * 128, 128)], ss_b)
        pltpu.async_copy(rbuf_b, relt_hbm.at[:, pl.ds(r1 * 128, 128)], rs_b)

        @pl.when(i0 + 2 < cnt)
        def _():
            pltpu.async_copy(t_hbm.at[sidx_v.at[i0 + 2]], rows_a, gs_a)

        return carry

    lax.fori_loop(0, cnt // 2, pair, 0)
    r_last = start + cnt - 1
    pltpu.make_async_copy(
        rows_b, g_hbm.at[pl.ds(r_last * 128, 128)], ss_b).wait()
    pltpu.make_async_copy(
        rbuf_b, relt_hbm.at[:, pl.ds(r_last * 128, 128)], rs_b).wait()


def _sc_gather(t, posf, src2d, dst2d):
    mesh = plsc.VectorSubcoreMesh(core_axis_name="c", subcore_axis_name="s")
    return pl.kernel(
        _sc_gather_body,
        out_type=[
            jax.ShapeDtypeStruct((E_EDGES, HID), jnp.float32),
            jax.ShapeDtypeStruct((3, E_EDGES), jnp.float32),
        ],
        mesh=mesh,
        scratch_types=[
            pltpu.VMEM((80, 128), jnp.int32),
            pltpu.VMEM((80, 128), jnp.int32),
            pltpu.VMEM((128, HID), jnp.float32),
            pltpu.VMEM((128, HID), jnp.float32),
            pltpu.VMEM((3, 128), jnp.float32),
            pltpu.VMEM((3, 128), jnp.float32),
            pltpu.VMEM((4 * N_NODES,), jnp.float32),
            pltpu.SemaphoreType.DMA,
            pltpu.SemaphoreType.DMA,
            pltpu.SemaphoreType.DMA,
            pltpu.SemaphoreType.DMA,
            pltpu.SemaphoreType.DMA,
            pltpu.SemaphoreType.DMA,
        ],
        compiler_params=pltpu.CompilerParams(needs_layout_passes=False),
    )(t, posf, src2d, dst2d)


def _sc_scatter_s_body(s_hbm, dst_hbm, z_hbm, sp_hbm,
                       didx_v, rows_a, rows_b, acc, ls_a, ls_b, as_a, as_b):
    c = lax.axis_index("c")
    s = lax.axis_index("s")
    w = s * _NC + c
    start = w * 80
    cnt = jnp.minimum(80, EROWS - start)

    pltpu.sync_copy(z_hbm, acc.at[pl.ds(s * _NPT, _NPT)])
    pltpu.sync_copy(dst_hbm.at[pl.ds(start, 80)], didx_v)
    plsc.subcore_barrier()

    pltpu.async_copy(s_hbm.at[pl.ds(start * 128, 128)], rows_a, ls_a)

    def pair(jj, carry):
        i0 = 2 * jj
        i1 = i0 + 1
        r0 = start + i0
        r1 = r0 + 1

        @pl.when(jj > 0)
        def _():
            pltpu.make_async_copy(
                rows_b, acc.at[didx_v.at[i0 - 1]], as_b).wait()

        pltpu.async_copy(s_hbm.at[pl.ds(r1 * 128, 128)], rows_b, ls_b)
        pltpu.make_async_copy(s_hbm.at[pl.ds(r0 * 128, 128)], rows_a, ls_a).wait()
        pltpu.async_copy(rows_a, acc.at[didx_v.at[i0]], as_a, add=True)
        pltpu.make_async_copy(s_hbm.at[pl.ds(r1 * 128, 128)], rows_b, ls_b).wait()
        pltpu.make_async_copy(rows_a, acc.at[didx_v.at[i0]], as_a).wait()
        pltpu.async_copy(rows_b, acc.at[didx_v.at[i1]], as_b, add=True)

        @pl.when(i0 + 2 < cnt)
        def _():
            pltpu.async_copy(s_hbm.at[pl.ds((r0 + 2) * 128, 128)], rows_a, ls_a)

        return carry

    lax.fori_loop(0, cnt // 2, pair, 0)
    pltpu.make_async_copy(rows_b, acc.at[didx_v.at[cnt - 1]], as_b).wait()
    plsc.subcore_barrier()
    pltpu.sync_copy(acc.at[pl.ds(s * _NPT, _NPT)],
                    sp_hbm.at[c].at[pl.ds(s * _NPT, _NPT)])


def _sc_scatter_s(smsg, dst2d, zeros):
    mesh = plsc.VectorSubcoreMesh(core_axis_name="c", subcore_axis_name="s")
    return pl.kernel(
        _sc_scatter_s_body,
        out_type=jax.ShapeDtypeStruct((2, _NPA, HID), jnp.float32),
        mesh=mesh,
        scratch_types=[
            pltpu.VMEM((80, 128), jnp.int32),
            pltpu.VMEM((128, HID), jnp.float32),
            pltpu.VMEM((128, HID), jnp.float32),
            pltpu.VMEM_SHARED((_NPA, HID), jnp.float32),
            pltpu.SemaphoreType.DMA,
            pltpu.SemaphoreType.DMA,
            pltpu.SemaphoreType.DMA,
            pltpu.SemaphoreType.DMA,
        ],
        compiler_params=pltpu.CompilerParams(needs_layout_passes=False),
    )(smsg, dst2d, zeros)


def _sc_scatter_v_body(vf_hbm, dst_hbm, vp_hbm,
                       didx_v, vbuf_a, vbuf_b, vacc, vs_a, vs_b):
    c = lax.axis_index("c")
    s = lax.axis_index("s")
    w = s * _NC + c
    start = w * 80
    cnt = jnp.minimum(80, EROWS - start)

    pltpu.sync_copy(dst_hbm.at[pl.ds(start, 80)], didx_v)

    def zero(i, carry):
        vacc[pl.ds(i * 16, 16)] = jnp.zeros((16,), jnp.float32)
        return carry

    lax.fori_loop(0, 3 * N_NODES // 16, zero, 0)

    def vec_add(i, vbuf):
        for g in range(8):
            di = didx_v[i, pl.ds(g * 16, 16)]
            for d in range(3):
                vals = vbuf[d, pl.ds(g * 16, 16)]
                off = jnp.full((16,), d * N_NODES, jnp.int32)
                plsc.addupdate_scatter(vacc, [di + off], vals)

    pltpu.async_copy(vf_hbm.at[:, pl.ds(start * 128, 128)], vbuf_a, vs_a)

    def pair(jj, carry):
        i0 = 2 * jj
        i1 = i0 + 1
        r0 = start + i0
        r1 = r0 + 1

        pltpu.async_copy(vf_hbm.at[:, pl.ds(r1 * 128, 128)], vbuf_b, vs_b)
        pltpu.make_async_copy(
            vf_hbm.at[:, pl.ds(r0 * 128, 128)], vbuf_a, vs_a).wait()
        vec_add(i0, vbuf_a)

        @pl.when(i0 + 2 < cnt)
        def _():
            pltpu.async_copy(vf_hbm.at[:, pl.ds((r0 + 2) * 128, 128)], vbuf_a, vs_a)

        pltpu.make_async_copy(
            vf_hbm.at[:, pl.ds(r1 * 128, 128)], vbuf_b, vs_b).wait()
        vec_add(i1, vbuf_b)
        return carry

    lax.fori_loop(0, cnt // 2, pair, 0)
    pltpu.sync_copy(vacc, vp_hbm.at[pl.ds(w * 3 * N_NODES, 3 * N_NODES)])


def _sc_scatter_v(vmsgt, dst2d):
    mesh = plsc.VectorSubcoreMesh(core_axis_name="c", subcore_axis_name="s")
    return pl.kernel(
        _sc_scatter_v_body,
        out_type=jax.ShapeDtypeStruct((_NW * 3 * N_NODES,), jnp.float32),
        mesh=mesh,
        scratch_types=[
            pltpu.VMEM((80, 128), jnp.int32),
            pltpu.VMEM((3, 128), jnp.float32),
            pltpu.VMEM((3, 128), jnp.float32),
            pltpu.VMEM((3 * N_NODES,), jnp.float32),
            pltpu.SemaphoreType.DMA,
            pltpu.SemaphoreType.DMA,
        ],
        compiler_params=pltpu.CompilerParams(needs_layout_passes=False),
    )(vmsgt, dst2d)


def _pad128(v):
    return jnp.concatenate([v, jnp.zeros(HID - v.shape[0], jnp.float32)])


def _layer_consts(p):
    ws = p['s_ln1_g'][:, None] * p['s_w1']
    wv = p['v_ln_g'][:, None] * p['v_w1']
    w1 = jnp.concatenate([p['a_w1'], ws, wv], axis=1)
    bs = p['s_b1'] + p['s_ln1_b'] @ p['s_w1']
    bv = p['v_b1'] + p['v_ln_b'] @ p['v_w1']
    ca = jnp.sum(p['a_w1'], axis=0)
    tail = jnp.zeros(HID, jnp.float32).at[0].set(p['a_b2'][0]).at[1].set(p['v_b2'][0])
    aux = jnp.stack([
        ca, p['a_b1'], bs, bv, p['a_w2'][:, 0], p['v_w2'][:, 0],
        p['s_b2'], p['s_ln2_g'], p['s_ln2_b'], tail,
        jnp.zeros(HID, jnp.float32), jnp.zeros(HID, jnp.float32),
        jnp.zeros(HID, jnp.float32), jnp.zeros(HID, jnp.float32),
        jnp.zeros(HID, jnp.float32), jnp.zeros(HID, jnp.float32),
    ])
    return w1, p['s_w2'], aux


def _node_aux(gb_row, p):
    a = jnp.zeros((8, HID), jnp.float32)
    a = a.at[0].set(gb_row)
    a = a.at[1].set(p['xn_g'])
    a = a.at[2].set(p['xn_b'])
    a = a.at[4:7, 0].set(p['pn_g'])
    a = a.at[4:7, 1].set(p['pn_b'])
    return a


def kernel(x, pos, edge_index, edge_attr, params):
    layers = params['layers']
    pos_t = jnp.concatenate([pos.T, jnp.zeros((1, N_NODES), jnp.float32)], axis=0)
    src = edge_index[0]
    dst = edge_index[1]
    src2d = jnp.pad(src.reshape(EROWS, 128), ((0, 60), (0, 0)))
    dst2d = jnp.pad(dst.reshape(EROWS, 128), ((0, 60), (0, 0)))
    zeros = jnp.zeros((_NPT, HID), jnp.float32)

    t, pt = _tc_prep(x, pos_t, _node_aux(jnp.zeros(HID, jnp.float32), layers[0]))
    for li, p in enumerate(layers):
        gat, relt = _sc_gather(t, pt.reshape(-1), src2d, dst2d)
        w1, sw2, aux = _layer_consts(p)
        smsg, vmsgt = _tc_edge(gat, relt, edge_attr, w1, sw2, aux)
        sp = _sc_scatter_s(smsg, dst2d, zeros)
        vp = _sc_scatter_v(vmsgt, dst2d)
        vp = vp.reshape(_NW, 3, N_NODES)
        if li + 1 < len(layers):
            t, pt = _tc_update(t, pt, sp, vp, p['g_w'],
                               _node_aux(p['g_b'], layers[li + 1]))
        else:
            faux = jnp.stack([p['g_b'], _pad128(params['e_b1']),
                              _pad128(params['e_b2']), jnp.zeros(HID, jnp.float32)])
            x_out, pos_out_t = _tc_final(t, pt, sp, vp, p['g_w'],
                                         params['e_w1'], params['e_w2'], faux)
    return (x_out, pos_out_t[:3, :].T)


# speedup vs baseline: 1.0735x; 1.0735x over previous
import functools

import jax
import jax.numpy as jnp
from jax import lax
from jax.experimental import pallas as pl
from jax.experimental.pallas import tpu as pltpu
from jax.experimental.pallas import tpu_sc as plsc

HID = 128
EDIM = 16
PPAD = 16
MW = HID + EDIM
N_NODES = 10000
E_EDGES = 320000
EROWS = E_EDGES // 128
EPS = 1e-6

EDGE_BLK = 6400
NODE_BLK = 2000

_NC = 2
_NS = 16
_NW = _NC * _NS
_RB = EROWS // _NW
_XTRA = EROWS - _RB * _NW
_NPA = 10112
_NPT = _NPA // _NS


def _silu(x):
    return x * jax.nn.sigmoid(x)


def _ln_x(x, g, b):
    m = jnp.sum(x, axis=-1, keepdims=True) / HID
    xc = x - m
    v = jnp.sum(xc * xc, axis=-1, keepdims=True) / HID
    return xc / jnp.sqrt(v + EPS) * g + b


def _ln_pos_t(p, g, b):
    rowmask = (lax.broadcasted_iota(jnp.int32, (4, 1), 0) < 3).astype(jnp.float32)
    m = jnp.sum(p, axis=0, keepdims=True) / 3.0
    pc = p - m
    pcm = pc * rowmask
    v = jnp.sum(pcm * pcm, axis=0, keepdims=True) / 3.0
    return pc / jnp.sqrt(v + EPS) * g + b


def _prep_body(x_ref, pt_ref, aux_ref, t_ref, p_ref):
    t_ref[...] = _ln_x(x_ref[...], aux_ref[1, :], aux_ref[2, :])
    p_ref[...] = _ln_pos_t(pt_ref[...], aux_ref[4:8, 0:1], aux_ref[4:8, 1:2])


def _edge_body(g_ref, rel_ref, attr_ref, w1_ref, sw2_ref, aux_ref,
               s_out_ref, v_out_ref):
    xj = g_ref[...]
    attr = attr_ref[...]

    ca = aux_ref[0, :]
    a_b1 = aux_ref[1, :]
    bs = aux_ref[2, :]
    bv = aux_ref[3, :]
    a_w2 = aux_ref[4, :]
    v_w2 = aux_ref[5, :]
    s_b2 = aux_ref[6, :]
    s2g = aux_ref[7, :]
    s2b = aux_ref[8, :]
    a_b2 = aux_ref[9, 0]
    v_b2 = aux_ref[9, 1]

    s1 = jnp.sum(xj, axis=-1, keepdims=True) + jnp.sum(attr, axis=-1, keepdims=True)
    m = s1 / MW
    xc = xj - m
    ac = attr - m
    var = (jnp.sum(xc * xc, axis=-1, keepdims=True)
           + jnp.sum(ac * ac, axis=-1, keepdims=True)) / MW
    sd = jnp.sqrt(var + EPS)
    inv = 1.0 / sd
    n = jnp.concatenate([xc * inv, ac * inv], axis=1)

    pre = jnp.dot(n, w1_ref[...], preferred_element_type=jnp.float32)
    pre_a = sd * pre[:, :HID] + m * ca + a_b1
    pre_s = pre[:, HID:2 * HID] + bs
    pre_v = pre[:, 2 * HID:] + bv

    a = jnp.sum(_silu(pre_a) * a_w2, axis=-1, keepdims=True) + a_b2
    attn = jax.nn.sigmoid(a)

    h = jnp.dot(_silu(pre_s), sw2_ref[...], preferred_element_type=jnp.float32) + s_b2
    h = _ln_x(h, s2g, s2b)
    s_out_ref[...] = h * attn

    rel_t = rel_ref[...]
    dist = jnp.maximum(
        jnp.sqrt(jnp.sum(rel_t * rel_t, axis=0, keepdims=True)), 1e-6)
    dims = (((0,), (1,)), ((), ()))
    a_row = lax.dot_general(a_w2[:, None], _silu(pre_a), dims,
                            preferred_element_type=jnp.float32) + a_b2
    vw_row = lax.dot_general(v_w2[:, None], _silu(pre_v), dims,
                             preferred_element_type=jnp.float32) + v_b2
    v_out_ref[...] = rel_t * (vw_row * jax.nn.sigmoid(a_row) / dist)


def _node_core(t_ref, pt_ref, p0_ref, p1_ref, vp_ref, gw_ref, gb):
    xln = t_ref[...]
    s_agg = p0_ref[...] + p1_ref[...]
    v_agg = jnp.sum(vp_ref[...], axis=0)
    v_agg = jnp.concatenate(
        [v_agg, jnp.zeros((1, v_agg.shape[1]), jnp.float32)], axis=0)
    gate = jax.nn.sigmoid(
        jnp.dot(xln, gw_ref[:HID, :], preferred_element_type=jnp.float32)
        + jnp.dot(s_agg, gw_ref[HID:, :], preferred_element_type=jnp.float32)
        + gb)
    x_new = xln * (1.0 - gate) + s_agg * gate
    pos_new = jnp.clip(pt_ref[...] + v_agg, -10.0, 10.0)
    return x_new, pos_new


def _update_body(t_ref, pt_ref, p0_ref, p1_ref, vp_ref, gw_ref, aux_ref,
                 t_out_ref, p_out_ref):
    x_new, pos_new = _node_core(t_ref, pt_ref, p0_ref, p1_ref, vp_ref, gw_ref,
                                aux_ref[0, :])
    t_out_ref[...] = _ln_x(x_new, aux_ref[1, :], aux_ref[2, :])
    p_out_ref[...] = _ln_pos_t(pos_new, aux_ref[4:8, 0:1], aux_ref[4:8, 1:2])


def _final_body(t_ref, pt_ref, p0_ref, p1_ref, vp_ref, gw_ref, ew1_ref,
                ew2_ref, aux_ref, x_out_ref, p_out_ref):
    x_new, pos_new = _node_core(t_ref, pt_ref, p0_ref, p1_ref, vp_ref, gw_ref,
                                aux_ref[0, :])
    y = jax.nn.relu(
        jnp.dot(x_new, ew1_ref[...], preferred_element_type=jnp.float32)
        + aux_ref[1, :])
    y = jnp.dot(y, ew2_ref[...], preferred_element_type=jnp.float32) + aux_ref[2, :]
    x_out_ref[...] = y
    p_out_ref[...] = pos_new


def _tc_prep(x, pos_t, aux):
    return pl.pallas_call(
        _prep_body,
        out_shape=[
            jax.ShapeDtypeStruct((N_NODES, HID), jnp.float32),
            jax.ShapeDtypeStruct((4, N_NODES), jnp.float32),
        ],
    )(x, pos_t, aux)


def _tc_edge(gat, rel, attr, w1, sw2, aux):
    grid = E_EDGES // EDGE_BLK
    return pl.pallas_call(
        _edge_body,
        grid=(grid,),
        in_specs=[
            pl.BlockSpec((EDGE_BLK, HID), lambda i: (i, 0)),
            pl.BlockSpec((3, EDGE_BLK), lambda i: (0, i)),
            pl.BlockSpec((EDGE_BLK, EDIM), lambda i: (i, 0)),
            pl.BlockSpec((MW, 3 * HID), lambda i: (0, 0)),
            pl.BlockSpec((HID, HID), lambda i: (0, 0)),
            pl.BlockSpec((16, HID), lambda i: (0, 0)),
        ],
        out_specs=[
            pl.BlockSpec((EDGE_BLK, HID), lambda i: (i, 0)),
            pl.BlockSpec((3, EDGE_BLK), lambda i: (0, i)),
        ],
        out_shape=[
            jax.ShapeDtypeStruct((E_EDGES, HID), jnp.float32),
            jax.ShapeDtypeStruct((3, E_EDGES), jnp.float32),
        ],
    )(gat, rel, attr, w1, sw2, aux)


def _tc_update(t, pt, sp, vp, gw, aux):
    return pl.pallas_call(
        _update_body,
        grid=(1,),
        in_specs=[
            pl.BlockSpec((N_NODES, HID), lambda i: (0, 0)),
            pl.BlockSpec((4, N_NODES), lambda i: (0, 0)),
            pl.BlockSpec((None, N_NODES, HID), lambda i: (0, 0, 0)),
            pl.BlockSpec((None, N_NODES, HID), lambda i: (1, 0, 0)),
            pl.BlockSpec((_NW, 3, N_NODES), lambda i: (0, 0, 0)),
            pl.BlockSpec((2 * HID, HID), lambda i: (0, 0)),
            pl.BlockSpec((8, HID), lambda i: (0, 0)),
        ],
        out_specs=[
            pl.BlockSpec((N_NODES, HID), lambda i: (0, 0)),
            pl.BlockSpec((4, N_NODES), lambda i: (0, 0)),
        ],
        out_shape=[
            jax.ShapeDtypeStruct((N_NODES, HID), jnp.float32),
            jax.ShapeDtypeStruct((4, N_NODES), jnp.float32),
        ],
    )(t, pt, sp, sp, vp, gw, aux)


def _tc_final(t, pt, sp, vp, gw, ew1, ew2, aux):
    return pl.pallas_call(
        _final_body,
        grid=(1,),
        in_specs=[
            pl.BlockSpec((N_NODES, HID), lambda i: (0, 0)),
            pl.BlockSpec((4, N_NODES), lambda i: (0, 0)),
            pl.BlockSpec((None, N_NODES, HID), lambda i: (0, 0, 0)),
            pl.BlockSpec((None, N_NODES, HID), lambda i: (1, 0, 0)),
            pl.BlockSpec((_NW, 3, N_NODES), lambda i: (0, 0, 0)),
            pl.BlockSpec((2 * HID, HID), lambda i: (0, 0)),
            pl.BlockSpec((HID, HID), lambda i: (0, 0)),
            pl.BlockSpec((HID, HID), lambda i: (0, 0)),
            pl.BlockSpec((4, HID), lambda i: (0, 0)),
        ],
        out_specs=[
            pl.BlockSpec((N_NODES, HID), lambda i: (0, 0)),
            pl.BlockSpec((4, N_NODES), lambda i: (0, 0)),
        ],
        out_shape=[
            jax.ShapeDtypeStruct((N_NODES, HID), jnp.float32),
            jax.ShapeDtypeStruct((4, N_NODES), jnp.float32),
        ],
    )(t, pt, sp, sp, vp, gw, ew1, ew2, aux)


def _worker_range(w):
    start = jnp.where(w < _XTRA, w * (_RB + 1), _XTRA * (_RB + 1) + (w - _XTRA) * _RB)
    cnt = jnp.where(w < _XTRA, _RB + 1, _RB)
    return start, cnt


def _sc_gather_body(t_hbm, pf_hbm, src_hbm, dst_hbm, g_hbm, relt_hbm,
                    sidx_v, didx_v, rows_a, rows_b, rbuf_a, rbuf_b, posf_v,
                    gs_a, gs_b, ss_a, ss_b, rs_a, rs_b):
    w = lax.axis_index("s") * _NC + lax.axis_index("c")
    start = w * 80
    cnt = jnp.minimum(80, EROWS - start)

    pltpu.sync_copy(pf_hbm, posf_v)
    pltpu.sync_copy(src_hbm.at[pl.ds(start, 80)], sidx_v)
    pltpu.sync_copy(dst_hbm.at[pl.ds(start, 80)], didx_v)

    def rel_compute(i, rbuf):
        for g in range(8):
            si = sidx_v[i, pl.ds(g * 16, 16)]
            di = didx_v[i, pl.ds(g * 16, 16)]
            for d in range(3):
                off = jnp.full((16,), d * N_NODES, jnp.int32)
                ps = plsc.load_gather(posf_v, [si + off])
                pd = plsc.load_gather(posf_v, [di + off])
                rbuf[d, pl.ds(g * 16, 16)] = ps - pd

    pltpu.async_copy(t_hbm.at[sidx_v.at[0]], rows_a, gs_a)

    def pair(jj, carry):
        i0 = 2 * jj
        i1 = i0 + 1
        r0 = start + i0
        r1 = r0 + 1

        @pl.when(jj > 0)
        def _():
            pltpu.make_async_copy(
                rows_b, g_hbm.at[pl.ds((r0 - 1) * 128, 128)], ss_b).wait()
            pltpu.make_async_copy(
                rbuf_b, relt_hbm.at[:, pl.ds((r0 - 1) * 128, 128)], rs_b).wait()

        pltpu.async_copy(t_hbm.at[sidx_v.at[i1]], rows_b, gs_b)
        pltpu.make_async_copy(t_hbm.at[sidx_v.at[i0]], rows_a, gs_a).wait()
        rel_compute(i0, rbuf_a)
        pltpu.async_copy(rows_a, g_hbm.at[pl.ds(r0 * 128, 128)], ss_a)
        pltpu.async_copy(rbuf_a, relt_hbm.at[:, pl.ds(r0 * 128, 128)], rs_a)
        pltpu.make_async_copy(t_hbm.at[sidx_v.at[i1]], rows_b, gs_b).wait()
        rel_compute(i1, rbuf_b)
        pltpu.make_async_copy(
            rows_a, g_hbm.at[pl.ds(r0 * 128, 128)], ss_a).wait()
        pltpu.make_async_copy(
            rbuf_a, relt_hbm.at[:, pl.ds(r0 * 128, 128)], rs_a).wait()
        pltpu.async_copy(rows_b, g_hbm.at[pl.ds(r1 * 128, 128)], ss_b)
        pltpu.async_copy(rbuf_b, relt_hbm.at[:, pl.ds(r1 * 128, 128)], rs_b)

        @pl.when(i0 + 2 < cnt)
        def _():
            pltpu.async_copy(t_hbm.at[sidx_v.at[i0 + 2]], rows_a, gs_a)

        return carry

    lax.fori_loop(0, cnt // 2, pair, 0)
    r_last = start + cnt - 1
    pltpu.make_async_copy(
        rows_b, g_hbm.at[pl.ds(r_last * 128, 128)], ss_b).wait()
    pltpu.make_async_copy(
        rbuf_b, relt_hbm.at[:, pl.ds(r_last * 128, 128)], rs_b).wait()


def _sc_gather(t, posf, src2d, dst2d):
    mesh = plsc.VectorSubcoreMesh(core_axis_name="c", subcore_axis_name="s")
    return pl.kernel(
        _sc_gather_body,
        out_type=[
            jax.ShapeDtypeStruct((E_EDGES, HID), jnp.float32),
            jax.ShapeDtypeStruct((3, E_EDGES), jnp.float32),
        ],
        mesh=mesh,
        scratch_types=[
            pltpu.VMEM((80, 128), jnp.int32),
            pltpu.VMEM((80, 128), jnp.int32),
            pltpu.VMEM((128, HID), jnp.float32),
            pltpu.VMEM((128, HID), jnp.float32),
            pltpu.VMEM((3, 128), jnp.float32),
            pltpu.VMEM((3, 128), jnp.float32),
            pltpu.VMEM((4 * N_NODES,), jnp.float32),
            pltpu.SemaphoreType.DMA,
            pltpu.SemaphoreType.DMA,
            pltpu.SemaphoreType.DMA,
            pltpu.SemaphoreType.DMA,
            pltpu.SemaphoreType.DMA,
            pltpu.SemaphoreType.DMA,
        ],
        compiler_params=pltpu.CompilerParams(needs_layout_passes=False),
    )(t, posf, src2d, dst2d)


def _sc_scatter_s_body(s_hbm, dst_hbm, z_hbm, sp_hbm,
                       didx_v, rows_a, rows_b, acc, ls_a, ls_b, as_a, as_b):
    c = lax.axis_index("c")
    s = lax.axis_index("s")
    w = s * _NC + c
    start = w * 80
    cnt = jnp.minimum(80, EROWS - start)

    pltpu.sync_copy(z_hbm, acc.at[pl.ds(s * _NPT, _NPT)])
    pltpu.sync_copy(dst_hbm.at[pl.ds(start, 80)], didx_v)
    plsc.subcore_barrier()

    pltpu.async_copy(s_hbm.at[pl.ds(start * 128, 128)], rows_a, ls_a)

    def pair(jj, carry):
        i0 = 2 * jj
        i1 = i0 + 1
        r0 = start + i0
        r1 = r0 + 1

        @pl.when(jj > 0)
        def _():
            pltpu.make_async_copy(
                rows_b, acc.at[didx_v.at[i0 - 1]], as_b).wait()

        pltpu.async_copy(s_hbm.at[pl.ds(r1 * 128, 128)], rows_b, ls_b)
        pltpu.make_async_copy(s_hbm.at[pl.ds(r0 * 128, 128)], rows_a, ls_a).wait()
        pltpu.async_copy(rows_a, acc.at[didx_v.at[i0]], as_a, add=True)
        pltpu.make_async_copy(s_hbm.at[pl.ds(r1 * 128, 128)], rows_b, ls_b).wait()
        pltpu.make_async_copy(rows_a, acc.at[didx_v.at[i0]], as_a).wait()
        pltpu.async_copy(rows_b, acc.at[didx_v.at[i1]], as_b, add=True)

        @pl.when(i0 + 2 < cnt)
        def _():
            pltpu.async_copy(s_hbm.at[pl.ds((r0 + 2) * 128, 128)], rows_a, ls_a)

        return carry

    lax.fori_loop(0, cnt // 2, pair, 0)
    pltpu.make_async_copy(rows_b, acc.at[didx_v.at[cnt - 1]], as_b).wait()
    plsc.subcore_barrier()
    pltpu.sync_copy(acc.at[pl.ds(s * _NPT, _NPT)],
                    sp_hbm.at[c].at[pl.ds(s * _NPT, _NPT)])


def _sc_scatter_s(smsg, dst2d, zeros):
    mesh = plsc.VectorSubcoreMesh(core_axis_name="c", subcore_axis_name="s")
    return pl.kernel(
        _sc_scatter_s_body,
        out_type=jax.ShapeDtypeStruct((2, _NPA, HID), jnp.float32),
        mesh=mesh,
        scratch_types=[
            pltpu.VMEM((80, 128), jnp.int32),
            pltpu.VMEM((128, HID), jnp.float32),
            pltpu.VMEM((128, HID), jnp.float32),
            pltpu.VMEM_SHARED((_NPA, HID), jnp.float32),
            pltpu.SemaphoreType.DMA,
            pltpu.SemaphoreType.DMA,
            pltpu.SemaphoreType.DMA,
            pltpu.SemaphoreType.DMA,
        ],
        compiler_params=pltpu.CompilerParams(needs_layout_passes=False),
    )(smsg, dst2d, zeros)


def _sc_scatter_v_body(vf_hbm, dst_hbm, vp_hbm,
                       didx_v, vbuf_a, vbuf_b, vacc, vs_a, vs_b):
    c = lax.axis_index("c")
    s = lax.axis_index("s")
    w = s * _NC + c
    start = w * 80
    cnt = jnp.minimum(80, EROWS - start)

    pltpu.sync_copy(dst_hbm.at[pl.ds(start, 80)], didx_v)

    def zero(i, carry):
        vacc[pl.ds(i * 16, 16)] = jnp.zeros((16,), jnp.float32)
        return carry

    lax.fori_loop(0, 3 * N_NODES // 16, zero, 0)

    def vec_add(i, vbuf):
        for g in range(8):
            di = didx_v[i, pl.ds(g * 16, 16)]
            for d in range(3):
                vals = vbuf[d, pl.ds(g * 16, 16)]
                off = jnp.full((16,), d * N_NODES, jnp.int32)
                plsc.addupdate_scatter(vacc, [di + off], vals)

    pltpu.async_copy(vf_hbm.at[:, pl.ds(start * 128, 128)], vbuf_a, vs_a)

    def pair(jj, carry):
        i0 = 2 * jj
        i1 = i0 + 1
        r0 = start + i0
        r1 = r0 + 1

        pltpu.async_copy(vf_hbm.at[:, pl.ds(r1 * 128, 128)], vbuf_b, vs_b)
        pltpu.make_async_copy(
            vf_hbm.at[:, pl.ds(r0 * 128, 128)], vbuf_a, vs_a).wait()
        vec_add(i0, vbuf_a)

        @pl.when(i0 + 2 < cnt)
        def _():
            pltpu.async_copy(vf_hbm.at[:, pl.ds((r0 + 2) * 128, 128)], vbuf_a, vs_a)

        pltpu.make_async_copy(
            vf_hbm.at[:, pl.ds(r1 * 128, 128)], vbuf_b, vs_b).wait()
        vec_add(i1, vbuf_b)
        return carry

    lax.fori_loop(0, cnt // 2, pair, 0)
    pltpu.sync_copy(vacc, vp_hbm.at[pl.ds(w * 3 * N_NODES, 3 * N_NODES)])


def _sc_scatter_v(vmsgt, dst2d):
    mesh = plsc.VectorSubcoreMesh(core_axis_name="c", subcore_axis_name="s")
    return pl.kernel(
        _sc_scatter_v_body,
        out_type=jax.ShapeDtypeStruct((_NW * 3 * N_NODES,), jnp.float32),
        mesh=mesh,
        scratch_types=[
            pltpu.VMEM((80, 128), jnp.int32),
            pltpu.VMEM((3, 128), jnp.float32),
            pltpu.VMEM((3, 128), jnp.float32),
            pltpu.VMEM((3 * N_NODES,), jnp.float32),
            pltpu.SemaphoreType.DMA,
            pltpu.SemaphoreType.DMA,
        ],
        compiler_params=pltpu.CompilerParams(needs_layout_passes=False),
    )(vmsgt, dst2d)


def _pad128(v):
    return jnp.concatenate([v, jnp.zeros(HID - v.shape[0], jnp.float32)])


def _layer_consts(p):
    ws = p['s_ln1_g'][:, None] * p['s_w1']
    wv = p['v_ln_g'][:, None] * p['v_w1']
    w1 = jnp.concatenate([p['a_w1'], ws, wv], axis=1)
    bs = p['s_b1'] + p['s_ln1_b'] @ p['s_w1']
    bv = p['v_b1'] + p['v_ln_b'] @ p['v_w1']
    ca = jnp.sum(p['a_w1'], axis=0)
    tail = jnp.zeros(HID, jnp.float32).at[0].set(p['a_b2'][0]).at[1].set(p['v_b2'][0])
    aux = jnp.stack([
        ca, p['a_b1'], bs, bv, p['a_w2'][:, 0], p['v_w2'][:, 0],
        p['s_b2'], p['s_ln2_g'], p['s_ln2_b'], tail,
        jnp.zeros(HID, jnp.float32), jnp.zeros(HID, jnp.float32),
        jnp.zeros(HID, jnp.float32), jnp.zeros(HID, jnp.float32),
        jnp.zeros(HID, jnp.float32), jnp.zeros(HID, jnp.float32),
    ])
    return w1, p['s_w2'], aux


def _node_aux(gb_row, p):
    a = jnp.zeros((8, HID), jnp.float32)
    a = a.at[0].set(gb_row)
    a = a.at[1].set(p['xn_g'])
    a = a.at[2].set(p['xn_b'])
    a = a.at[4:7, 0].set(p['pn_g'])
    a = a.at[4:7, 1].set(p['pn_b'])
    return a


def kernel(x, pos, edge_index, edge_attr, params):
    layers = params['layers']
    pos_t = jnp.concatenate([pos.T, jnp.zeros((1, N_NODES), jnp.float32)], axis=0)
    src = edge_index[0]
    dst = edge_index[1]
    src2d = jnp.pad(src.reshape(EROWS, 128), ((0, 60), (0, 0)))
    dst2d = jnp.pad(dst.reshape(EROWS, 128), ((0, 60), (0, 0)))
    zeros = jnp.zeros((_NPT, HID), jnp.float32)

    t, pt = _tc_prep(x, pos_t, _node_aux(jnp.zeros(HID, jnp.float32), layers[0]))
    for li, p in enumerate(layers):
        gat, relt = _sc_gather(t, pt.reshape(-1), src2d, dst2d)
        w1, sw2, aux = _layer_consts(p)
        smsg, vmsgt = _tc_edge(gat, relt, edge_attr, w1, sw2, aux)
        sp = _sc_scatter_s(smsg, dst2d, zeros)
        vp = _sc_scatter_v(vmsgt, dst2d)
        vp = vp.reshape(_NW, 3, N_NODES)
        if li + 1 < len(layers):
            t, pt = _tc_update(t, pt, sp, vp, p['g_w'],
                               _node_aux(p['g_b'], layers[li + 1]))
        else:
            faux = jnp.stack([p['g_b'], _pad128(params['e_b1']),
                              _pad128(params['e_b2']), jnp.zeros(HID, jnp.float32)])
            x_out, pos_out_t = _tc_final(t, pt, sp, vp, p['g_w'],
                                         params['e_w1'], params['e_w2'], faux)
    return (x_out, pos_out_t[:3, :].T)


# speedup vs baseline: 1.0781x; 1.0042x over previous
import functools

import jax
import jax.numpy as jnp
from jax import lax
from jax.experimental import pallas as pl
from jax.experimental.pallas import tpu as pltpu
from jax.experimental.pallas import tpu_sc as plsc

HID = 128
EDIM = 16
PPAD = 16
MW = HID + EDIM
N_NODES = 10000
E_EDGES = 320000
EROWS = E_EDGES // 128
EPS = 1e-6

EDGE_BLK = 6400
NODE_BLK = 2000

_NC = 2
_NS = 16
_NW = _NC * _NS
_RB = EROWS // _NW
_XTRA = EROWS - _RB * _NW
_NPA = 10112
_NPT = _NPA // _NS


def _silu(x):
    return x * jax.nn.sigmoid(x)


def _ln_x(x, g, b):
    m = jnp.sum(x, axis=-1, keepdims=True) / HID
    xc = x - m
    v = jnp.sum(xc * xc, axis=-1, keepdims=True) / HID
    inv = 1.0 / jnp.sqrt(v + EPS)
    return xc * inv * g + b


def _ln_pos_t(p, g, b):
    rowmask = (lax.broadcasted_iota(jnp.int32, (4, 1), 0) < 3).astype(jnp.float32)
    m = jnp.sum(p, axis=0, keepdims=True) / 3.0
    pc = p - m
    pcm = pc * rowmask
    v = jnp.sum(pcm * pcm, axis=0, keepdims=True) / 3.0
    inv = 1.0 / jnp.sqrt(v + EPS)
    return pc * inv * g + b


def _prep_body(x_ref, pt_ref, aux_ref, t_ref, p_ref):
    t_ref[...] = _ln_x(x_ref[...], aux_ref[1, :], aux_ref[2, :])
    p_ref[...] = _ln_pos_t(pt_ref[...], aux_ref[4:8, 0:1], aux_ref[4:8, 1:2])


def _edge_body(g_ref, rel_ref, attr_ref, w1_ref, sw2_ref, aux_ref,
               s_out_ref, v_out_ref):
    xj = g_ref[...]
    attr = attr_ref[...]

    ca = aux_ref[0, :]
    a_b1 = aux_ref[1, :]
    bs = aux_ref[2, :]
    bv = aux_ref[3, :]
    a_w2 = aux_ref[4, :]
    v_w2 = aux_ref[5, :]
    s_b2 = aux_ref[6, :]
    s2g = aux_ref[7, :]
    s2b = aux_ref[8, :]
    a_b2 = aux_ref[9, 0]
    v_b2 = aux_ref[9, 1]

    s1 = jnp.sum(xj, axis=-1, keepdims=True) + jnp.sum(attr, axis=-1, keepdims=True)
    m = s1 / MW
    xc = xj - m
    ac = attr - m
    var = (jnp.sum(xc * xc, axis=-1, keepdims=True)
           + jnp.sum(ac * ac, axis=-1, keepdims=True)) / MW
    sd = jnp.sqrt(var + EPS)
    inv = 1.0 / sd
    n = jnp.concatenate([xc * inv, ac * inv], axis=1)

    pre = jnp.dot(n, w1_ref[...], preferred_element_type=jnp.float32)
    pre_a = sd * pre[:, :HID] + m * ca + a_b1
    pre_s = pre[:, HID:2 * HID] + bs
    pre_v = pre[:, 2 * HID:] + bv

    a = jnp.sum(_silu(pre_a) * a_w2, axis=-1, keepdims=True) + a_b2
    attn = jax.nn.sigmoid(a)

    h = jnp.dot(_silu(pre_s), sw2_ref[...], preferred_element_type=jnp.float32) + s_b2
    h = _ln_x(h, s2g, s2b)
    s_out_ref[...] = h * attn

    rel_t = rel_ref[...]
    dist = jnp.maximum(
        jnp.sqrt(jnp.sum(rel_t * rel_t, axis=0, keepdims=True)), 1e-6)
    dims = (((0,), (1,)), ((), ()))
    a_row = lax.dot_general(a_w2[:, None], _silu(pre_a), dims,
                            preferred_element_type=jnp.float32) + a_b2
    vw_row = lax.dot_general(v_w2[:, None], _silu(pre_v), dims,
                             preferred_element_type=jnp.float32) + v_b2
    v_out_ref[...] = rel_t * (vw_row * jax.nn.sigmoid(a_row) * (1.0 / dist))


def _node_core(t_ref, pt_ref, p0_ref, p1_ref, vp_ref, gw_ref, gb):
    xln = t_ref[...]
    s_agg = p0_ref[...] + p1_ref[...]
    v_agg = jnp.sum(vp_ref[...], axis=0)
    v_agg = jnp.concatenate(
        [v_agg, jnp.zeros((1, v_agg.shape[1]), jnp.float32)], axis=0)
    gate = jax.nn.sigmoid(
        jnp.dot(xln, gw_ref[:HID, :], preferred_element_type=jnp.float32)
        + jnp.dot(s_agg, gw_ref[HID:, :], preferred_element_type=jnp.float32)
        + gb)
    x_new = xln * (1.0 - gate) + s_agg * gate
    pos_new = jnp.clip(pt_ref[...] + v_agg, -10.0, 10.0)
    return x_new, pos_new


def _update_body(t_ref, pt_ref, p0_ref, p1_ref, vp_ref, gw_ref, aux_ref,
                 t_out_ref, p_out_ref):
    x_new, pos_new = _node_core(t_ref, pt_ref, p0_ref, p1_ref, vp_ref, gw_ref,
                                aux_ref[0, :])
    t_out_ref[...] = _ln_x(x_new, aux_ref[1, :], aux_ref[2, :])
    p_out_ref[...] = _ln_pos_t(pos_new, aux_ref[4:8, 0:1], aux_ref[4:8, 1:2])


def _final_body(t_ref, pt_ref, p0_ref, p1_ref, vp_ref, gw_ref, ew1_ref,
                ew2_ref, aux_ref, x_out_ref, p_out_ref):
    x_new, pos_new = _node_core(t_ref, pt_ref, p0_ref, p1_ref, vp_ref, gw_ref,
                                aux_ref[0, :])
    y = jax.nn.relu(
        jnp.dot(x_new, ew1_ref[...], preferred_element_type=jnp.float32)
        + aux_ref[1, :])
    y = jnp.dot(y, ew2_ref[...], preferred_element_type=jnp.float32) + aux_ref[2, :]
    x_out_ref[...] = y
    p_out_ref[...] = pos_new


def _tc_prep(x, pos_t, aux):
    return pl.pallas_call(
        _prep_body,
        out_shape=[
            jax.ShapeDtypeStruct((N_NODES, HID), jnp.float32),
            jax.ShapeDtypeStruct((4, N_NODES), jnp.float32),
        ],
    )(x, pos_t, aux)


def _tc_edge(gat, rel, attr, w1, sw2, aux):
    grid = E_EDGES // EDGE_BLK
    return pl.pallas_call(
        _edge_body,
        grid=(grid,),
        in_specs=[
            pl.BlockSpec((EDGE_BLK, HID), lambda i: (i, 0)),
            pl.BlockSpec((3, EDGE_BLK), lambda i: (0, i)),
            pl.BlockSpec((EDGE_BLK, EDIM), lambda i: (i, 0)),
            pl.BlockSpec((MW, 3 * HID), lambda i: (0, 0)),
            pl.BlockSpec((HID, HID), lambda i: (0, 0)),
            pl.BlockSpec((16, HID), lambda i: (0, 0)),
        ],
        out_specs=[
            pl.BlockSpec((EDGE_BLK, HID), lambda i: (i, 0)),
            pl.BlockSpec((3, EDGE_BLK), lambda i: (0, i)),
        ],
        out_shape=[
            jax.ShapeDtypeStruct((E_EDGES, HID), jnp.float32),
            jax.ShapeDtypeStruct((3, E_EDGES), jnp.float32),
        ],
    )(gat, rel, attr, w1, sw2, aux)


def _tc_update(t, pt, sp, vp, gw, aux):
    return pl.pallas_call(
        _update_body,
        grid=(1,),
        in_specs=[
            pl.BlockSpec((N_NODES, HID), lambda i: (0, 0)),
            pl.BlockSpec((4, N_NODES), lambda i: (0, 0)),
            pl.BlockSpec((None, N_NODES, HID), lambda i: (0, 0, 0)),
            pl.BlockSpec((None, N_NODES, HID), lambda i: (1, 0, 0)),
            pl.BlockSpec((_NW, 3, N_NODES), lambda i: (0, 0, 0)),
            pl.BlockSpec((2 * HID, HID), lambda i: (0, 0)),
            pl.BlockSpec((8, HID), lambda i: (0, 0)),
        ],
        out_specs=[
            pl.BlockSpec((N_NODES, HID), lambda i: (0, 0)),
            pl.BlockSpec((4, N_NODES), lambda i: (0, 0)),
        ],
        out_shape=[
            jax.ShapeDtypeStruct((N_NODES, HID), jnp.float32),
            jax.ShapeDtypeStruct((4, N_NODES), jnp.float32),
        ],
    )(t, pt, sp, sp, vp, gw, aux)


def _tc_final(t, pt, sp, vp, gw, ew1, ew2, aux):
    return pl.pallas_call(
        _final_body,
        grid=(1,),
        in_specs=[
            pl.BlockSpec((N_NODES, HID), lambda i: (0, 0)),
            pl.BlockSpec((4, N_NODES), lambda i: (0, 0)),
            pl.BlockSpec((None, N_NODES, HID), lambda i: (0, 0, 0)),
            pl.BlockSpec((None, N_NODES, HID), lambda i: (1, 0, 0)),
            pl.BlockSpec((_NW, 3, N_NODES), lambda i: (0, 0, 0)),
            pl.BlockSpec((2 * HID, HID), lambda i: (0, 0)),
            pl.BlockSpec((HID, HID), lambda i: (0, 0)),
            pl.BlockSpec((HID, HID), lambda i: (0, 0)),
            pl.BlockSpec((4, HID), lambda i: (0, 0)),
        ],
        out_specs=[
            pl.BlockSpec((N_NODES, HID), lambda i: (0, 0)),
            pl.BlockSpec((4, N_NODES), lambda i: (0, 0)),
        ],
        out_shape=[
            jax.ShapeDtypeStruct((N_NODES, HID), jnp.float32),
            jax.ShapeDtypeStruct((4, N_NODES), jnp.float32),
        ],
    )(t, pt, sp, sp, vp, gw, ew1, ew2, aux)


def _worker_range(w):
    start = jnp.where(w < _XTRA, w * (_RB + 1), _XTRA * (_RB + 1) + (w - _XTRA) * _RB)
    cnt = jnp.where(w < _XTRA, _RB + 1, _RB)
    return start, cnt


def _sc_gather_body(t_hbm, pf_hbm, src_hbm, dst_hbm, g_hbm, relt_hbm,
                    sidx_v, didx_v, rows_a, rows_b, rbuf_a, rbuf_b, posf_v,
                    gs_a, gs_b, ss_a, ss_b, rs_a, rs_b):
    w = lax.axis_index("s") * _NC + lax.axis_index("c")
    start = w * 80
    cnt = jnp.minimum(80, EROWS - start)

    pltpu.sync_copy(pf_hbm, posf_v)
    pltpu.sync_copy(src_hbm.at[pl.ds(start, 80)], sidx_v)
    pltpu.sync_copy(dst_hbm.at[pl.ds(start, 80)], didx_v)

    def rel_compute(i, rbuf):
        for g in range(8):
            si = sidx_v[i, pl.ds(g * 16, 16)]
            di = didx_v[i, pl.ds(g * 16, 16)]
            for d in range(3):
                off = jnp.full((16,), d * N_NODES, jnp.int32)
                ps = plsc.load_gather(posf_v, [si + off])
                pd = plsc.load_gather(posf_v, [di + off])
                rbuf[d, pl.ds(g * 16, 16)] = ps - pd

    pltpu.async_copy(t_hbm.at[sidx_v.at[0]], rows_a, gs_a)

    def pair(jj, carry):
        i0 = 2 * jj
        i1 = i0 + 1
        r0 = start + i0
        r1 = r0 + 1

        @pl.when(jj > 0)
        def _():
            pltpu.make_async_copy(
                rows_b, g_hbm.at[pl.ds((r0 - 1) * 128, 128)], ss_b).wait()
            pltpu.make_async_copy(
                rbuf_b, relt_hbm.at[:, pl.ds((r0 - 1) * 128, 128)], rs_b).wait()

        pltpu.async_copy(t_hbm.at[sidx_v.at[i1]], rows_b, gs_b)
        pltpu.make_async_copy(t_hbm.at[sidx_v.at[i0]], rows_a, gs_a).wait()
        rel_compute(i0, rbuf_a)
        pltpu.async_copy(rows_a, g_hbm.at[pl.ds(r0 * 128, 128)], ss_a)
        pltpu.async_copy(rbuf_a, relt_hbm.at[:, pl.ds(r0 * 128, 128)], rs_a)
        pltpu.make_async_copy(t_hbm.at[sidx_v.at[i1]], rows_b, gs_b).wait()
        rel_compute(i1, rbuf_b)
        pltpu.make_async_copy(
            rows_a, g_hbm.at[pl.ds(r0 * 128, 128)], ss_a).wait()
        pltpu.make_async_copy(
            rbuf_a, relt_hbm.at[:, pl.ds(r0 * 128, 128)], rs_a).wait()
        pltpu.async_copy(rows_b, g_hbm.at[pl.ds(r1 * 128, 128)], ss_b)
        pltpu.async_copy(rbuf_b, relt_hbm.at[:, pl.ds(r1 * 128, 128)], rs_b)

        @pl.when(i0 + 2 < cnt)
        def _():
            pltpu.async_copy(t_hbm.at[sidx_v.at[i0 + 2]], rows_a, gs_a)

        return carry

    lax.fori_loop(0, cnt // 2, pair, 0)
    r_last = start + cnt - 1
    pltpu.make_async_copy(
        rows_b, g_hbm.at[pl.ds(r_last * 128, 128)], ss_b).wait()
    pltpu.make_async_copy(
        rbuf_b, relt_hbm.at[:, pl.ds(r_last * 128, 128)], rs_b).wait()


def _sc_gather(t, posf, src2d, dst2d):
    mesh = plsc.VectorSubcoreMesh(core_axis_name="c", subcore_axis_name="s")
    return pl.kernel(
        _sc_gather_body,
        out_type=[
            jax.ShapeDtypeStruct((E_EDGES, HID), jnp.float32),
            jax.ShapeDtypeStruct((3, E_EDGES), jnp.float32),
        ],
        mesh=mesh,
        scratch_types=[
            pltpu.VMEM((80, 128), jnp.int32),
            pltpu.VMEM((80, 128), jnp.int32),
            pltpu.VMEM((128, HID), jnp.float32),
            pltpu.VMEM((128, HID), jnp.float32),
            pltpu.VMEM((3, 128), jnp.float32),
            pltpu.VMEM((3, 128), jnp.float32),
            pltpu.VMEM((4 * N_NODES,), jnp.float32),
            pltpu.SemaphoreType.DMA,
            pltpu.SemaphoreType.DMA,
            pltpu.SemaphoreType.DMA,
            pltpu.SemaphoreType.DMA,
            pltpu.SemaphoreType.DMA,
            pltpu.SemaphoreType.DMA,
        ],
        compiler_params=pltpu.CompilerParams(needs_layout_passes=False),
    )(t, posf, src2d, dst2d)


def _sc_scatter_s_body(s_hbm, dst_hbm, z_hbm, sp_hbm,
                       didx_v, rows_a, rows_b, acc, ls_a, ls_b, as_a, as_b):
    c = lax.axis_index("c")
    s = lax.axis_index("s")
    w = s * _NC + c
    start = w * 80
    cnt = jnp.minimum(80, EROWS - start)

    pltpu.sync_copy(z_hbm, acc.at[pl.ds(s * _NPT, _NPT)])
    pltpu.sync_copy(dst_hbm.at[pl.ds(start, 80)], didx_v)
    plsc.subcore_barrier()

    pltpu.async_copy(s_hbm.at[pl.ds(start * 128, 128)], rows_a, ls_a)

    def pair(jj, carry):
        i0 = 2 * jj
        i1 = i0 + 1
        r0 = start + i0
        r1 = r0 + 1

        @pl.when(jj > 0)
        def _():
            pltpu.make_async_copy(
                rows_b, acc.at[didx_v.at[i0 - 1]], as_b).wait()

        pltpu.async_copy(s_hbm.at[pl.ds(r1 * 128, 128)], rows_b, ls_b)
        pltpu.make_async_copy(s_hbm.at[pl.ds(r0 * 128, 128)], rows_a, ls_a).wait()
        pltpu.async_copy(rows_a, acc.at[didx_v.at[i0]], as_a, add=True)
        pltpu.make_async_copy(s_hbm.at[pl.ds(r1 * 128, 128)], rows_b, ls_b).wait()
        pltpu.make_async_copy(rows_a, acc.at[didx_v.at[i0]], as_a).wait()
        pltpu.async_copy(rows_b, acc.at[didx_v.at[i1]], as_b, add=True)

        @pl.when(i0 + 2 < cnt)
        def _():
            pltpu.async_copy(s_hbm.at[pl.ds((r0 + 2) * 128, 128)], rows_a, ls_a)

        return carry

    lax.fori_loop(0, cnt // 2, pair, 0)
    pltpu.make_async_copy(rows_b, acc.at[didx_v.at[cnt - 1]], as_b).wait()
    plsc.subcore_barrier()
    pltpu.sync_copy(acc.at[pl.ds(s * _NPT, _NPT)],
                    sp_hbm.at[c].at[pl.ds(s * _NPT, _NPT)])


def _sc_scatter_s(smsg, dst2d, zeros):
    mesh = plsc.VectorSubcoreMesh(core_axis_name="c", subcore_axis_name="s")
    return pl.kernel(
        _sc_scatter_s_body,
        out_type=jax.ShapeDtypeStruct((2, _NPA, HID), jnp.float32),
        mesh=mesh,
        scratch_types=[
            pltpu.VMEM((80, 128), jnp.int32),
            pltpu.VMEM((128, HID), jnp.float32),
            pltpu.VMEM((128, HID), jnp.float32),
            pltpu.VMEM_SHARED((_NPA, HID), jnp.float32),
            pltpu.SemaphoreType.DMA,
            pltpu.SemaphoreType.DMA,
            pltpu.SemaphoreType.DMA,
            pltpu.SemaphoreType.DMA,
        ],
        compiler_params=pltpu.CompilerParams(needs_layout_passes=False),
    )(smsg, dst2d, zeros)


def _sc_scatter_v_body(vf_hbm, dst_hbm, vp_hbm,
                       didx_v, vbuf_a, vbuf_b, vacc, vs_a, vs_b):
    c = lax.axis_index("c")
    s = lax.axis_index("s")
    w = s * _NC + c
    start = w * 80
    cnt = jnp.minimum(80, EROWS - start)

    pltpu.sync_copy(dst_hbm.at[pl.ds(start, 80)], didx_v)

    def zero(i, carry):
        vacc[pl.ds(i * 16, 16)] = jnp.zeros((16,), jnp.float32)
        return carry

    lax.fori_loop(0, 3 * N_NODES // 16, zero, 0)

    def vec_add(i, vbuf):
        for g in range(8):
            di = didx_v[i, pl.ds(g * 16, 16)]
            for d in range(3):
                vals = vbuf[d, pl.ds(g * 16, 16)]
                off = jnp.full((16,), d * N_NODES, jnp.int32)
                plsc.addupdate_scatter(vacc, [di + off], vals)

    pltpu.async_copy(vf_hbm.at[:, pl.ds(start * 128, 128)], vbuf_a, vs_a)

    def pair(jj, carry):
        i0 = 2 * jj
        i1 = i0 + 1
        r0 = start + i0
        r1 = r0 + 1

        pltpu.async_copy(vf_hbm.at[:, pl.ds(r1 * 128, 128)], vbuf_b, vs_b)
        pltpu.make_async_copy(
            vf_hbm.at[:, pl.ds(r0 * 128, 128)], vbuf_a, vs_a).wait()
        vec_add(i0, vbuf_a)

        @pl.when(i0 + 2 < cnt)
        def _():
            pltpu.async_copy(vf_hbm.at[:, pl.ds((r0 + 2) * 128, 128)], vbuf_a, vs_a)

        pltpu.make_async_copy(
            vf_hbm.at[:, pl.ds(r1 * 128, 128)], vbuf_b, vs_b).wait()
        vec_add(i1, vbuf_b)
        return carry

    lax.fori_loop(0, cnt // 2, pair, 0)
    pltpu.sync_copy(vacc, vp_hbm.at[pl.ds(w * 3 * N_NODES, 3 * N_NODES)])


def _sc_scatter_v(vmsgt, dst2d):
    mesh = plsc.VectorSubcoreMesh(core_axis_name="c", subcore_axis_name="s")
    return pl.kernel(
        _sc_scatter_v_body,
        out_type=jax.ShapeDtypeStruct((_NW * 3 * N_NODES,), jnp.float32),
        mesh=mesh,
        scratch_types=[
            pltpu.VMEM((80, 128), jnp.int32),
            pltpu.VMEM((3, 128), jnp.float32),
            pltpu.VMEM((3, 128), jnp.float32),
            pltpu.VMEM((3 * N_NODES,), jnp.float32),
            pltpu.SemaphoreType.DMA,
            pltpu.SemaphoreType.DMA,
        ],
        compiler_params=pltpu.CompilerParams(needs_layout_passes=False),
    )(vmsgt, dst2d)


def _pad128(v):
    return jnp.concatenate([v, jnp.zeros(HID - v.shape[0], jnp.float32)])


def _layer_consts(p):
    ws = p['s_ln1_g'][:, None] * p['s_w1']
    wv = p['v_ln_g'][:, None] * p['v_w1']
    w1 = jnp.concatenate([p['a_w1'], ws, wv], axis=1)
    bs = p['s_b1'] + p['s_ln1_b'] @ p['s_w1']
    bv = p['v_b1'] + p['v_ln_b'] @ p['v_w1']
    ca = jnp.sum(p['a_w1'], axis=0)
    tail = jnp.zeros(HID, jnp.float32).at[0].set(p['a_b2'][0]).at[1].set(p['v_b2'][0])
    aux = jnp.stack([
        ca, p['a_b1'], bs, bv, p['a_w2'][:, 0], p['v_w2'][:, 0],
        p['s_b2'], p['s_ln2_g'], p['s_ln2_b'], tail,
        jnp.zeros(HID, jnp.float32), jnp.zeros(HID, jnp.float32),
        jnp.zeros(HID, jnp.float32), jnp.zeros(HID, jnp.float32),
        jnp.zeros(HID, jnp.float32), jnp.zeros(HID, jnp.float32),
    ])
    return w1, p['s_w2'], aux


def _node_aux(gb_row, p):
    a = jnp.zeros((8, HID), jnp.float32)
    a = a.at[0].set(gb_row)
    a = a.at[1].set(p['xn_g'])
    a = a.at[2].set(p['xn_b'])
    a = a.at[4:7, 0].set(p['pn_g'])
    a = a.at[4:7, 1].set(p['pn_b'])
    return a


def kernel(x, pos, edge_index, edge_attr, params):
    layers = params['layers']
    pos_t = jnp.concatenate([pos.T, jnp.zeros((1, N_NODES), jnp.float32)], axis=0)
    src = edge_index[0]
    dst = edge_index[1]
    src2d = jnp.pad(src.reshape(EROWS, 128), ((0, 60), (0, 0)))
    dst2d = jnp.pad(dst.reshape(EROWS, 128), ((0, 60), (0, 0)))
    zeros = jnp.zeros((_NPT, HID), jnp.float32)

    t, pt = _tc_prep(x, pos_t, _node_aux(jnp.zeros(HID, jnp.float32), layers[0]))
    for li, p in enumerate(layers):
        gat, relt = _sc_gather(t, pt.reshape(-1), src2d, dst2d)
        w1, sw2, aux = _layer_consts(p)
        smsg, vmsgt = _tc_edge(gat, relt, edge_attr, w1, sw2, aux)
        sp = _sc_scatter_s(smsg, dst2d, zeros)
        vp = _sc_scatter_v(vmsgt, dst2d)
        vp = vp.reshape(_NW, 3, N_NODES)
        if li + 1 < len(layers):
            t, pt = _tc_update(t, pt, sp, vp, p['g_w'],
                               _node_aux(p['g_b'], layers[li + 1]))
        else:
            faux = jnp.stack([p['g_b'], _pad128(params['e_b1']),
                              _pad128(params['e_b2']), jnp.zeros(HID, jnp.float32)])
            x_out, pos_out_t = _tc_final(t, pt, sp, vp, p['g_w'],
                                         params['e_w1'], params['e_w2'], faux)
    return (x_out, pos_out_t[:3, :].T)
